# baseline (speedup 1.0000x reference)
import functools

import jax
import jax.numpy as jnp
from jax import lax
from jax.experimental import pallas as pl
from jax.experimental.pallas import tpu as pltpu


BM = 128
FF = 1024


def _ffn_body(be_ref, valid_ref, xg_ref, w1_ref, w3_ref, w2_ref, out_ref,
              acc_ref, w1s_ref, w3s_ref, w2s_ref):
    f = pl.program_id(0)
    m = pl.program_id(1)
    nf = pl.num_programs(0)

    @pl.when(f == 0)
    def _init():
        acc_ref[pl.ds(m * BM, BM), :] = jnp.zeros((BM, acc_ref.shape[1]),
                                                  jnp.float32)

    new_w = jnp.logical_or(m == 0,
                           be_ref[m] != be_ref[jnp.maximum(m - 1, 0)])

    @pl.when(new_w)
    def _cast():
        w1s_ref[...] = w1_ref[0].astype(jnp.bfloat16)
        w3s_ref[...] = w3_ref[0].astype(jnp.bfloat16)
        w2s_ref[...] = w2_ref[0].astype(jnp.bfloat16)

    @pl.when(valid_ref[m] > 0)
    def _compute():
        xb = xg_ref[...].astype(jnp.bfloat16)
        dn = (((1,), (1,)), ((), ()))
        x1 = lax.dot_general(xb, w1s_ref[...], dn,
                             preferred_element_type=jnp.float32)
        x3 = lax.dot_general(xb, w3s_ref[...], dn,
                             preferred_element_type=jnp.float32)
        h = (x1 * jax.nn.sigmoid(x1) * x3).astype(jnp.bfloat16)
        contrib = lax.dot_general(h, w2s_ref[...], dn,
                                  preferred_element_type=jnp.float32)
        acc_ref[pl.ds(m * BM, BM), :] += contrib

    @pl.when(f == nf - 1)
    def _write():
        out_ref[...] = acc_ref[pl.ds(m * BM, BM), :]


def _grouped_ffn(xg, w1, w3, w2, be, valid, cap_rows):
    num_e, inter, dim = w1.shape
    m_blocks = cap_rows // BM
    f_blocks = inter // FF
    grid_spec = pltpu.PrefetchScalarGridSpec(
        num_scalar_prefetch=2,
        grid=(f_blocks, m_blocks),
        in_specs=[
            pl.BlockSpec((BM, dim), lambda f, m, be, va: (m, 0)),
            pl.BlockSpec((1, FF, dim), lambda f, m, be, va: (be[m], f, 0)),
            pl.BlockSpec((1, FF, dim), lambda f, m, be, va: (be[m], f, 0)),
            pl.BlockSpec((1, dim, FF), lambda f, m, be, va: (be[m], 0, f)),
        ],
        out_specs=pl.BlockSpec((BM, dim), lambda f, m, be, va: (m, 0)),
        scratch_shapes=[
            pltpu.VMEM((cap_rows, dim), jnp.float32),
            pltpu.VMEM((FF, dim), jnp.bfloat16),
            pltpu.VMEM((FF, dim), jnp.bfloat16),
            pltpu.VMEM((dim, FF), jnp.bfloat16),
        ],
    )
    return pl.pallas_call(
        _ffn_body,
        grid_spec=grid_spec,
        out_shape=jax.ShapeDtypeStruct((cap_rows, dim), jnp.float32),
        compiler_params=pltpu.CompilerParams(
            dimension_semantics=("arbitrary", "arbitrary")),
    )(be, valid, xg, w1, w3, w2)


def kernel(x, expert_indices, w1, w2, w3):
    seq_len, dim = x.shape
    top_k = expert_indices.shape[1]
    num_e = w1.shape[0]
    p = seq_len * top_k
    cap_rows = p + num_e * BM
    m_blocks = cap_rows // BM

    e_flat = expert_indices.reshape(-1).astype(jnp.int32)
    oh = (e_flat[:, None] == jnp.arange(num_e, dtype=jnp.int32)[None, :])
    oh = oh.astype(jnp.int32)
    counts = oh.sum(0)
    nb = (counts + BM - 1) // BM
    starts_blk = jnp.concatenate(
        [jnp.zeros((1,), jnp.int32), jnp.cumsum(nb)[:-1].astype(jnp.int32)])
    rank = (jnp.cumsum(oh, axis=0) * oh).sum(1) - 1
    pos = starts_blk[e_flat] * BM + rank
    total_blk = nb.sum()
    bids = jnp.arange(m_blocks, dtype=jnp.int32)
    be = jnp.searchsorted(starts_blk, bids, side="right").astype(jnp.int32) - 1
    e_last = (jnp.searchsorted(starts_blk, total_blk - 1, side="right")
              .astype(jnp.int32) - 1)
    be = jnp.where(bids < total_blk, be, e_last).astype(jnp.int32)
    valid = (bids < total_blk).astype(jnp.int32)
    tok = jnp.arange(p, dtype=jnp.int32) // top_k
    tok_padded = jnp.zeros((cap_rows,), jnp.int32).at[pos].set(tok)

    xg = x[tok_padded]

    y = _grouped_ffn(xg, w1, w3, w2, be, valid, cap_rows)

    out = y[pos]
    return out.reshape(seq_len, top_k, dim)


# speedup vs baseline: 1.0707x; 1.0707x over previous
import functools

import jax
import jax.numpy as jnp
from jax import lax
from jax.experimental import pallas as pl
from jax.experimental.pallas import tpu as pltpu
from jax.experimental.pallas import tpu_sc as plsc


BM = 128
FF = 1024


def _sc_scatter_rows(x, pos_k, cap_rows):
    s, d = x.shape
    info = plsc.get_sparse_core_info()
    nw = info.num_cores * info.num_subcores
    c = s // nw
    mesh = plsc.VectorSubcoreMesh(core_axis_name="c", subcore_axis_name="s")

    @functools.partial(
        pl.kernel, mesh=mesh,
        out_type=jax.ShapeDtypeStruct((cap_rows, d), jnp.float32),
        scratch_types=[
            pltpu.VMEM((c, d), jnp.float32),
            pltpu.VMEM((c,), jnp.int32),
            pltpu.VMEM((c,), jnp.int32),
            pltpu.SemaphoreType.DMA,
        ],
    )
    def body(x_hbm, pos_hbm, xg_hbm, rows_v, idx0_v, idx1_v, sem):
        wid = lax.axis_index("s") * info.num_cores + lax.axis_index("c")
        base = wid * c
        pltpu.sync_copy(x_hbm.at[pl.ds(base, c)], rows_v)
        pltpu.sync_copy(pos_hbm.at[0, pl.ds(base, c)], idx0_v)
        pltpu.sync_copy(pos_hbm.at[1, pl.ds(base, c)], idx1_v)
        cp0 = pltpu.async_copy(rows_v, xg_hbm.at[idx0_v], sem)
        cp1 = pltpu.async_copy(rows_v, xg_hbm.at[idx1_v], sem)
        cp0.wait()
        cp1.wait()

    return body(x, pos_k)


def _sc_gather_rows(table, idx, chunk):
    b = idx.shape[0]
    d = table.shape[1]
    info = plsc.get_sparse_core_info()
    nw = info.num_cores * info.num_subcores
    b_per_w = b // nw
    n_ch = b_per_w // chunk
    mesh = plsc.VectorSubcoreMesh(core_axis_name="c", subcore_axis_name="s")

    @functools.partial(
        pl.kernel, mesh=mesh,
        out_type=jax.ShapeDtypeStruct((b, d), jnp.float32),
        scratch_types=[
            pltpu.VMEM((chunk, d), jnp.float32),
            pltpu.VMEM((chunk,), jnp.int32),
            pltpu.SemaphoreType.DMA,
        ],
    )
    def body(table_hbm, idx_hbm, out_hbm, rows_v, idx_v, sem):
        wid = lax.axis_index("s") * info.num_cores + lax.axis_index("c")
        base = wid * b_per_w
        for j in range(n_ch):
            off = base + j * chunk
            pltpu.sync_copy(idx_hbm.at[pl.ds(off, chunk)], idx_v)
            pltpu.async_copy(table_hbm.at[idx_v], rows_v, sem).wait()
            pltpu.sync_copy(rows_v, out_hbm.at[pl.ds(off, chunk)])

    return body(table, idx)


def _ffn_body(be_ref, valid_ref, xg_ref, w1_ref, w3_ref, w2_ref, out_ref,
              acc_ref, w1s_ref, w3s_ref, w2s_ref):
    f = pl.program_id(0)
    m = pl.program_id(1)
    nf = pl.num_programs(0)

    @pl.when(f == 0)
    def _init():
        acc_ref[pl.ds(m * BM, BM), :] = jnp.zeros((BM, acc_ref.shape[1]),
                                                  jnp.float32)

    new_w = jnp.logical_or(m == 0,
                           be_ref[m] != be_ref[jnp.maximum(m - 1, 0)])

    @pl.when(new_w)
    def _cast():
        w1s_ref[...] = w1_ref[0].astype(jnp.bfloat16)
        w3s_ref[...] = w3_ref[0].astype(jnp.bfloat16)
        w2s_ref[...] = w2_ref[0].astype(jnp.bfloat16)

    @pl.when(valid_ref[m] > 0)
    def _compute():
        xb = xg_ref[...].astype(jnp.bfloat16)
        dn = (((1,), (1,)), ((), ()))
        x1 = lax.dot_general(xb, w1s_ref[...], dn,
                             preferred_element_type=jnp.float32)
        x3 = lax.dot_general(xb, w3s_ref[...], dn,
                             preferred_element_type=jnp.float32)
        h = (x1 * jax.nn.sigmoid(x1) * x3).astype(jnp.bfloat16)
        contrib = lax.dot_general(h, w2s_ref[...], dn,
                                  preferred_element_type=jnp.float32)
        acc_ref[pl.ds(m * BM, BM), :] += contrib

    @pl.when(f == nf - 1)
    def _write():
        out_ref[...] = acc_ref[pl.ds(m * BM, BM), :]


def _grouped_ffn(xg, w1, w3, w2, be, valid, cap_rows):
    num_e, inter, dim = w1.shape
    m_blocks = cap_rows // BM
    f_blocks = inter // FF
    grid_spec = pltpu.PrefetchScalarGridSpec(
        num_scalar_prefetch=2,
        grid=(f_blocks, m_blocks),
        in_specs=[
            pl.BlockSpec((BM, dim), lambda f, m, be, va: (m, 0)),
            pl.BlockSpec((1, FF, dim), lambda f, m, be, va: (be[m], f, 0)),
            pl.BlockSpec((1, FF, dim), lambda f, m, be, va: (be[m], f, 0)),
            pl.BlockSpec((1, dim, FF), lambda f, m, be, va: (be[m], 0, f)),
        ],
        out_specs=pl.BlockSpec((BM, dim), lambda f, m, be, va: (m, 0)),
        scratch_shapes=[
            pltpu.VMEM((cap_rows, dim), jnp.float32),
            pltpu.VMEM((FF, dim), jnp.bfloat16),
            pltpu.VMEM((FF, dim), jnp.bfloat16),
            pltpu.VMEM((dim, FF), jnp.bfloat16),
        ],
    )
    return pl.pallas_call(
        _ffn_body,
        grid_spec=grid_spec,
        out_shape=jax.ShapeDtypeStruct((cap_rows, dim), jnp.float32),
        compiler_params=pltpu.CompilerParams(
            dimension_semantics=("arbitrary", "arbitrary")),
    )(be, valid, xg, w1, w3, w2)


def kernel(x, expert_indices, w1, w2, w3):
    seq_len, dim = x.shape
    top_k = expert_indices.shape[1]
    num_e = w1.shape[0]
    p = seq_len * top_k
    cap_rows = p + num_e * BM
    m_blocks = cap_rows // BM

    e_flat = expert_indices.reshape(-1).astype(jnp.int32)
    oh = (e_flat[:, None] == jnp.arange(num_e, dtype=jnp.int32)[None, :])
    oh = oh.astype(jnp.int32)
    counts = oh.sum(0)
    nb = (counts + BM - 1) // BM
    starts_blk = jnp.concatenate(
        [jnp.zeros((1,), jnp.int32), jnp.cumsum(nb)[:-1].astype(jnp.int32)])
    rank = (jnp.cumsum(oh, axis=0) * oh).sum(1) - 1
    pos = starts_blk[e_flat] * BM + rank
    total_blk = nb.sum()
    bids = jnp.arange(m_blocks, dtype=jnp.int32)
    be = jnp.searchsorted(starts_blk, bids, side="right").astype(jnp.int32) - 1
    e_last = (jnp.searchsorted(starts_blk, total_blk - 1, side="right")
              .astype(jnp.int32) - 1)
    be = jnp.where(bids < total_blk, be, e_last).astype(jnp.int32)
    valid = (bids < total_blk).astype(jnp.int32)
    pos_k = pos.reshape(seq_len, top_k).T

    xg = _sc_scatter_rows(x, pos_k, cap_rows)

    y = _grouped_ffn(xg, w1, w3, w2, be, valid, cap_rows)

    out = _sc_gather_rows(y, pos, 64)
    return out.reshape(seq_len, top_k, dim)
